# Initial kernel scaffold; baseline (speedup 1.0000x reference)
import jax
import jax.numpy as jnp
from jax.experimental import pallas as pl


def kernel(x, edge_index, W1l, W1r, b1, W2l, W2r, b2, Wout, bout):
    raise NotImplementedError("write your pallas kernel here")


# speedup vs baseline: 2.7596x; 2.7596x over previous
import functools

import jax
import jax.numpy as jnp
from jax import lax
from jax.experimental import pallas as pl
from jax.experimental.pallas import tpu as pltpu
from jax.experimental.pallas import tpu_sc as plsc

N = 10000
NPAD = 10240
NE = 160000
NSUB = 16
NCORE = 2
CH = 128
NCHUNK = 80
DCHUNK = NCHUNK // NCORE
EPS = NCHUNK * CH
ROWS_PER_SUB = NPAD // NSUB
ZR = 64
D = 256
DH = 128


def _seg_sum_body(table, srcE, dstE, z128,
                  S, src_v, dst_v, rows_v, accum, sem):
    c = lax.axis_index("c")
    s = lax.axis_index("s")
    row0 = s * ROWS_PER_SUB

    def zero_step(i, carry):
        pltpu.sync_copy(z128, accum.at[pl.ds(row0 + i * ZR, ZR)])
        return carry

    lax.fori_loop(0, ROWS_PER_SUB // ZR, zero_step, 0)
    pltpu.sync_copy(srcE.at[s], src_v)
    pltpu.sync_copy(dstE.at[s], dst_v)
    plsc.subcore_barrier()

    def chunk(j, carry):
        pltpu.async_copy(table.at[c].at[src_v.at[j]], rows_v, sem).wait()
        pltpu.sync_copy(rows_v, accum.at[dst_v.at[j]], add=True)
        return carry

    lax.fori_loop(0, NCHUNK, chunk, 0)
    plsc.subcore_barrier()
    pltpu.sync_copy(accum.at[pl.ds(row0, ROWS_PER_SUB)],
                    S.at[c, pl.ds(row0, ROWS_PER_SUB)])


def _make_seg_sum():
    mesh = plsc.VectorSubcoreMesh(core_axis_name="c", subcore_axis_name="s")
    return pl.kernel(
        _seg_sum_body,
        out_type=[jax.ShapeDtypeStruct((NCORE, NPAD, DH), jnp.float32)],
        mesh=mesh,
        scratch_types=[
            pltpu.VMEM((NCHUNK, CH), jnp.int32),
            pltpu.VMEM((NCHUNK, CH), jnp.int32),
            pltpu.VMEM((CH, DH), jnp.float32),
            pltpu.VMEM_SHARED((NPAD, DH), jnp.float32),
            pltpu.SemaphoreType.DMA,
        ])


def _deg_body(dstE2, z128, cones, degP, dst_v, ones_v, deg_sp, sem):
    c = lax.axis_index("c")
    s = lax.axis_index("s")
    row0 = s * ROWS_PER_SUB

    def zero_step(i, carry):
        pltpu.sync_copy(z128, deg_sp.at[pl.ds(row0 + i * ZR, ZR)])
        return carry

    lax.fori_loop(0, ROWS_PER_SUB // ZR, zero_step, 0)
    pltpu.sync_copy(cones, ones_v)
    pltpu.sync_copy(dstE2.at[s, c], dst_v)
    plsc.subcore_barrier()

    def chunk(j, carry):
        pltpu.sync_copy(ones_v, deg_sp.at[dst_v.at[j]], add=True)
        return carry

    lax.fori_loop(0, DCHUNK, chunk, 0)
    plsc.subcore_barrier()
    pltpu.sync_copy(deg_sp.at[pl.ds(row0, ROWS_PER_SUB)],
                    degP.at[c, pl.ds(row0, ROWS_PER_SUB)])


def _make_deg():
    mesh = plsc.VectorSubcoreMesh(core_axis_name="c", subcore_axis_name="s")
    return pl.kernel(
        _deg_body,
        out_type=[jax.ShapeDtypeStruct((NCORE, NPAD, DH), jnp.float32)],
        mesh=mesh,
        scratch_types=[
            pltpu.VMEM((DCHUNK, CH), jnp.int32),
            pltpu.VMEM((CH, DH), jnp.float32),
            pltpu.VMEM_SHARED((NPAD, DH), jnp.float32),
            pltpu.SemaphoreType.DMA,
        ])


_TC_R = 1024


def _layer1_body(s_ref, deg_ref, x_ref, wl_ref, wr_ref, b_ref, o_ref):
    deg = deg_ref[0, :, 0:1] + deg_ref[1, :, 0:1]
    inv = 1.0 / jnp.maximum(deg, 1.0)
    mean = jnp.concatenate([s_ref[0], s_ref[1]], axis=1) * inv
    h = jnp.dot(mean, wl_ref[...], preferred_element_type=jnp.float32)
    h = h + jnp.dot(x_ref[...], wr_ref[...], preferred_element_type=jnp.float32)
    h = jnp.maximum(h + b_ref[...], 0.0)
    o_ref[0] = h[:, :DH]
    o_ref[1] = h[:, DH:]


def _layer2_body(s_ref, deg_ref, h_ref, wl_ref, wr_ref, b_ref,
                 wo_ref, bo_ref, o_ref):
    deg = deg_ref[0, :, 0:1] + deg_ref[1, :, 0:1]
    inv = 1.0 / jnp.maximum(deg, 1.0)
    mean = jnp.concatenate([s_ref[0], s_ref[1]], axis=1) * inv
    h1 = jnp.concatenate([h_ref[0], h_ref[1]], axis=1)
    h2 = jnp.dot(mean, wl_ref[...], preferred_element_type=jnp.float32)
    h2 = h2 + jnp.dot(h1, wr_ref[...], preferred_element_type=jnp.float32)
    h2 = h2 + b_ref[...]
    o_ref[...] = (jnp.dot(h2, wo_ref[...], preferred_element_type=jnp.float32)
                  + bo_ref[...])


def _tc_layer1(S1, degP, x_pad, W1l, W1r, b1):
    grid = NPAD // _TC_R
    return pl.pallas_call(
        _layer1_body,
        grid=(grid,),
        in_specs=[
            pl.BlockSpec((NCORE, _TC_R, DH), lambda i: (0, i, 0)),
            pl.BlockSpec((NCORE, _TC_R, DH), lambda i: (0, i, 0)),
            pl.BlockSpec((_TC_R, D), lambda i: (i, 0)),
            pl.BlockSpec((D, D), lambda i: (0, 0)),
            pl.BlockSpec((D, D), lambda i: (0, 0)),
            pl.BlockSpec((1, D), lambda i: (0, 0)),
        ],
        out_specs=pl.BlockSpec((NCORE, _TC_R, DH), lambda i: (0, i, 0)),
        out_shape=jax.ShapeDtypeStruct((NCORE, NPAD, DH), jnp.float32),
    )(S1, degP, x_pad, W1l, W1r, b1)


def _tc_layer2(S2, degP, h1s, W2l, W2r, b2, Wout, bout):
    grid = NPAD // _TC_R
    return pl.pallas_call(
        _layer2_body,
        grid=(grid,),
        in_specs=[
            pl.BlockSpec((NCORE, _TC_R, DH), lambda i: (0, i, 0)),
            pl.BlockSpec((NCORE, _TC_R, DH), lambda i: (0, i, 0)),
            pl.BlockSpec((NCORE, _TC_R, DH), lambda i: (0, i, 0)),
            pl.BlockSpec((D, D), lambda i: (0, 0)),
            pl.BlockSpec((D, D), lambda i: (0, 0)),
            pl.BlockSpec((1, D), lambda i: (0, 0)),
            pl.BlockSpec((D, 128), lambda i: (0, 0)),
            pl.BlockSpec((1, 128), lambda i: (0, 0)),
        ],
        out_specs=pl.BlockSpec((_TC_R, 128), lambda i: (i, 0)),
        out_shape=jax.ShapeDtypeStruct((NPAD, 128), jnp.float32),
    )(S2, degP, h1s, W2l, W2r, b2, Wout, bout)


def kernel(x, edge_index, W1l, W1r, b1, W2l, W2r, b2, Wout, bout):
    src = edge_index[0]
    dst = edge_index[1]

    x_pad = jnp.pad(x, ((0, NPAD - N), (0, 0)))
    xs = jnp.stack([x_pad[:, :DH], x_pad[:, DH:]])

    pad_e = NSUB * EPS - NE
    srcp = jnp.pad(src, (0, pad_e))
    dstp = jnp.pad(dst, (0, pad_e), constant_values=N)
    srcE = srcp.reshape(NSUB, NCHUNK, CH)
    dstE = dstp.reshape(NSUB, NCHUNK, CH)
    dstE2 = dstp.reshape(NSUB, NCORE, DCHUNK, CH)

    z128 = jnp.zeros((ZR, DH), jnp.float32)
    cones = jnp.ones((CH, DH), jnp.float32)

    degP, = _make_deg()(dstE2, z128, cones)
    S1, = _make_seg_sum()(xs, srcE, dstE, z128)
    h1s = _tc_layer1(S1, degP, x_pad, W1l, W1r, b1.reshape(1, D))

    S2, = _make_seg_sum()(h1s, srcE, dstE, z128)
    out = _tc_layer2(S2, degP, h1s, W2l, W2r, b2.reshape(1, D),
                     Wout, bout.reshape(1, 128))
    return out[:N]


# speedup vs baseline: 3.6419x; 1.3197x over previous
import functools

import jax
import jax.numpy as jnp
from jax import lax
from jax.experimental import pallas as pl
from jax.experimental.pallas import tpu as pltpu
from jax.experimental.pallas import tpu_sc as plsc

N = 10000
NPAD = 10240
NE = 160000
NSUB = 16
NCORE = 2
CH = 128
NCHUNK = 80
DCHUNK = NCHUNK // NCORE
EPS = NCHUNK * CH
ROWS_PER_SUB = NPAD // NSUB
ZR = 64
D = 256
DH = 128


HCHUNK = NCHUNK // 2


def _seg_sum_body(table, srcE, dstE, z128, cones, flag, S, degP,
                  src_v, dst_v, r0, r1, g0, g1, s0, s1, flag_s, accum):
    c = lax.axis_index("c")
    sid = lax.axis_index("s")
    row0 = sid * ROWS_PER_SUB
    rows = [r0, r1]
    gsem = [g0, g1]
    ssem = [s0, s1]

    def zero(i, carry):
        pltpu.sync_copy(z128, accum.at[pl.ds(row0 + i * ZR, ZR)])
        return carry

    lax.fori_loop(0, ROWS_PER_SUB // ZR, zero, 0)

    def start_g(j, b):
        pltpu.async_copy(table.at[c].at[src_v.at[j]], rows[b], gsem[b])

    def wait_g(j, b):
        pltpu.make_async_copy(table.at[c].at[src_v.at[j]], rows[b],
                              gsem[b]).wait()

    def start_s(j, b):
        pltpu.async_copy(rows[b], accum.at[dst_v.at[j]], ssem[b], add=True)

    def wait_s(j, b):
        pltpu.make_async_copy(rows[b], accum.at[dst_v.at[j]],
                              ssem[b]).wait()

    for h in range(2):
        pltpu.sync_copy(srcE.at[sid, pl.ds(h * HCHUNK, HCHUNK)], src_v)
        pltpu.sync_copy(dstE.at[sid, pl.ds(h * HCHUNK, HCHUNK)], dst_v)
        if h == 0:
            plsc.subcore_barrier()
        start_g(0, 0)

        def pair(g, carry):
            for b in range(2):
                j = 2 * g + b
                bn = (b + 1) % 2

                @pl.when(j >= 1)
                def _():
                    wait_s(lax.max(j - 1, 0), bn)

                @pl.when(j + 1 < HCHUNK)
                def _():
                    start_g(j + 1, bn)

                wait_g(j, b)
                start_s(j, b)
            return carry

        lax.fori_loop(0, HCHUNK // 2, pair, 0)
        wait_s(HCHUNK - 1, (HCHUNK - 1) % 2)
    plsc.subcore_barrier()
    pltpu.sync_copy(accum.at[pl.ds(row0, ROWS_PER_SUB)],
                    S.at[c, pl.ds(row0, ROWS_PER_SUB)])

    pltpu.sync_copy(flag, flag_s)
    flag_v = flag_s[...]

    @pl.when(flag_v[0] > 0)
    def _deg_phase():
        lax.fori_loop(0, ROWS_PER_SUB // ZR, zero, 0)
        pltpu.sync_copy(cones, rows[0])
        pltpu.sync_copy(dstE.at[sid, pl.ds(c * DCHUNK, DCHUNK)], dst_v)
        plsc.subcore_barrier()

        def ones_pair(g, carry):
            for b in range(2):
                j = 2 * g + b
                pltpu.async_copy(rows[0], accum.at[dst_v.at[j]], ssem[b],
                                 add=True)
            for b in range(2):
                pltpu.make_async_copy(rows[0], accum.at[dst_v.at[2 * g + b]],
                                      ssem[b]).wait()
            return carry

        lax.fori_loop(0, DCHUNK // 2, ones_pair, 0)
        plsc.subcore_barrier()
        pltpu.sync_copy(accum.at[pl.ds(row0, ROWS_PER_SUB)],
                        degP.at[c, pl.ds(row0, ROWS_PER_SUB)])


_SEG_SUM = None


def _make_seg_sum():
    global _SEG_SUM
    if _SEG_SUM is not None:
        return _SEG_SUM
    mesh = plsc.VectorSubcoreMesh(core_axis_name="c", subcore_axis_name="s")
    _SEG_SUM = pl.kernel(
        _seg_sum_body,
        out_type=[jax.ShapeDtypeStruct((NCORE, NPAD, DH), jnp.float32),
                  jax.ShapeDtypeStruct((NCORE, NPAD, DH), jnp.float32)],
        mesh=mesh,
        scratch_types=[
            pltpu.VMEM((HCHUNK, CH), jnp.int32),
            pltpu.VMEM((HCHUNK, CH), jnp.int32),
            pltpu.VMEM((CH, DH), jnp.float32),
            pltpu.VMEM((CH, DH), jnp.float32),
            pltpu.SemaphoreType.DMA,
            pltpu.SemaphoreType.DMA,
            pltpu.SemaphoreType.DMA,
            pltpu.SemaphoreType.DMA,
            pltpu.VMEM((16,), jnp.int32),
            pltpu.VMEM_SHARED((NPAD, DH), jnp.float32),
        ])
    return _SEG_SUM


_TC_R = 1024


def _layer1_body(s_ref, deg_ref, x_ref, wl_ref, wr_ref, b_ref, o_ref):
    deg = deg_ref[0, :, 0:1] + deg_ref[1, :, 0:1]
    inv = 1.0 / jnp.maximum(deg, 1.0)
    mean = jnp.concatenate([s_ref[0], s_ref[1]], axis=1) * inv
    h = jnp.dot(mean, wl_ref[...], preferred_element_type=jnp.float32)
    h = h + jnp.dot(x_ref[...], wr_ref[...], preferred_element_type=jnp.float32)
    h = jnp.maximum(h + b_ref[...], 0.0)
    o_ref[0] = h[:, :DH]
    o_ref[1] = h[:, DH:]


def _layer2_body(s_ref, deg_ref, h_ref, wl_ref, wr_ref, b_ref,
                 wo_ref, bo_ref, o_ref):
    deg = deg_ref[0, :, 0:1] + deg_ref[1, :, 0:1]
    inv = 1.0 / jnp.maximum(deg, 1.0)
    mean = jnp.concatenate([s_ref[0], s_ref[1]], axis=1) * inv
    h1 = jnp.concatenate([h_ref[0], h_ref[1]], axis=1)
    h2 = jnp.dot(mean, wl_ref[...], preferred_element_type=jnp.float32)
    h2 = h2 + jnp.dot(h1, wr_ref[...], preferred_element_type=jnp.float32)
    h2 = h2 + b_ref[...]
    o_ref[...] = (jnp.dot(h2, wo_ref[...], preferred_element_type=jnp.float32)
                  + bo_ref[...])


def _tc_layer1(S1, degP, x_pad, W1l, W1r, b1):
    grid = NPAD // _TC_R
    return pl.pallas_call(
        _layer1_body,
        grid=(grid,),
        in_specs=[
            pl.BlockSpec((NCORE, _TC_R, DH), lambda i: (0, i, 0)),
            pl.BlockSpec((NCORE, _TC_R, DH), lambda i: (0, i, 0)),
            pl.BlockSpec((_TC_R, D), lambda i: (i, 0)),
            pl.BlockSpec((D, D), lambda i: (0, 0)),
            pl.BlockSpec((D, D), lambda i: (0, 0)),
            pl.BlockSpec((1, D), lambda i: (0, 0)),
        ],
        out_specs=pl.BlockSpec((NCORE, _TC_R, DH), lambda i: (0, i, 0)),
        out_shape=jax.ShapeDtypeStruct((NCORE, NPAD, DH), jnp.float32),
    )(S1, degP, x_pad, W1l, W1r, b1)


def _tc_layer2(S2, degP, h1s, W2l, W2r, b2, Wout, bout):
    grid = NPAD // _TC_R
    return pl.pallas_call(
        _layer2_body,
        grid=(grid,),
        in_specs=[
            pl.BlockSpec((NCORE, _TC_R, DH), lambda i: (0, i, 0)),
            pl.BlockSpec((NCORE, _TC_R, DH), lambda i: (0, i, 0)),
            pl.BlockSpec((NCORE, _TC_R, DH), lambda i: (0, i, 0)),
            pl.BlockSpec((D, D), lambda i: (0, 0)),
            pl.BlockSpec((D, D), lambda i: (0, 0)),
            pl.BlockSpec((1, D), lambda i: (0, 0)),
            pl.BlockSpec((D, 128), lambda i: (0, 0)),
            pl.BlockSpec((1, 128), lambda i: (0, 0)),
        ],
        out_specs=pl.BlockSpec((_TC_R, 128), lambda i: (i, 0)),
        out_shape=jax.ShapeDtypeStruct((NPAD, 128), jnp.float32),
    )(S2, degP, h1s, W2l, W2r, b2, Wout, bout)


def kernel(x, edge_index, W1l, W1r, b1, W2l, W2r, b2, Wout, bout):
    src = edge_index[0]
    dst = edge_index[1]

    x_pad = jnp.pad(x, ((0, NPAD - N), (0, 0)))
    xs = jnp.stack([x_pad[:, :DH], x_pad[:, DH:]])

    pad_e = NSUB * EPS - NE
    srcp = jnp.pad(src, (0, pad_e))
    dstp = jnp.pad(dst, (0, pad_e), constant_values=N)
    srcE = srcp.reshape(NSUB, NCHUNK, CH)
    dstE = dstp.reshape(NSUB, NCHUNK, CH)

    z128 = jnp.zeros((ZR, DH), jnp.float32)
    cones = jnp.ones((CH, DH), jnp.float32)

    one_f = jnp.ones((16,), jnp.int32)
    zero_f = jnp.zeros((16,), jnp.int32)
    S1, degP = _make_seg_sum()(xs, srcE, dstE, z128, cones, one_f)
    h1s = _tc_layer1(S1, degP, x_pad, W1l, W1r, b1.reshape(1, D))

    S2, _ = _make_seg_sum()(h1s, srcE, dstE, z128, cones, zero_f)
    out = _tc_layer2(S2, degP, h1s, W2l, W2r, b2.reshape(1, D),
                     Wout, bout.reshape(1, 128))
    return out[:N]


# speedup vs baseline: 3.6915x; 1.0136x over previous
import functools

import jax
import jax.numpy as jnp
from jax import lax
from jax.experimental import pallas as pl
from jax.experimental.pallas import tpu as pltpu
from jax.experimental.pallas import tpu_sc as plsc

N = 10000
NPAD = 10240
NE = 160000
NSUB = 16
NCORE = 2
CH = 128
NCHUNK = 80
DCHUNK = NCHUNK // NCORE
EPS = NCHUNK * CH
ROWS_PER_SUB = NPAD // NSUB
ZR = 128
D = 256
DH = 128


HCHUNK = NCHUNK // 2


def _seg_sum_body(table, srcE, dstE, z128, cones, flag, S, degP,
                  src_v, dst_v, r0, r1, g0, g1, s0, s1, flag_s, accum):
    c = lax.axis_index("c")
    sid = lax.axis_index("s")
    row0 = sid * ROWS_PER_SUB
    rows = [r0, r1]
    gsem = [g0, g1]
    ssem = [s0, s1]

    def zero(i, carry):
        pltpu.sync_copy(z128, accum.at[pl.ds(row0 + i * ZR, ZR)])
        return carry

    def zero_all():
        nz = ROWS_PER_SUB // ZR

        def zpair(i, carry):
            d0 = pltpu.async_copy(z128, accum.at[pl.ds(row0 + 2 * i * ZR, ZR)],
                                  g0)
            d1 = pltpu.async_copy(
                z128, accum.at[pl.ds(row0 + (2 * i + 1) * ZR, ZR)], g1)
            d0.wait()
            d1.wait()
            return carry

        lax.fori_loop(0, nz // 2, zpair, 0)
        if nz % 2:
            pltpu.sync_copy(z128, accum.at[pl.ds(row0 + (nz - 1) * ZR, ZR)])

    zero_all()

    def start_g(j, b):
        pltpu.async_copy(table.at[c].at[src_v.at[j]], rows[b], gsem[b])

    def wait_g(j, b):
        pltpu.make_async_copy(table.at[c].at[src_v.at[j]], rows[b],
                              gsem[b]).wait()

    def start_s(j, b):
        pltpu.async_copy(rows[b], accum.at[dst_v.at[j]], ssem[b], add=True)

    def wait_s(j, b):
        pltpu.make_async_copy(rows[b], accum.at[dst_v.at[j]],
                              ssem[b]).wait()

    for h in range(2):
        pltpu.sync_copy(srcE.at[sid, pl.ds(h * HCHUNK, HCHUNK)], src_v)
        pltpu.sync_copy(dstE.at[sid, pl.ds(h * HCHUNK, HCHUNK)], dst_v)
        if h == 0:
            plsc.subcore_barrier()
        start_g(0, 0)

        def pair(g, carry):
            for b in range(2):
                j = 2 * g + b
                bn = (b + 1) % 2

                @pl.when(j >= 1)
                def _():
                    wait_s(lax.max(j - 1, 0), bn)

                @pl.when(j + 1 < HCHUNK)
                def _():
                    start_g(j + 1, bn)

                wait_g(j, b)
                start_s(j, b)
            return carry

        lax.fori_loop(0, HCHUNK // 2, pair, 0)
        wait_s(HCHUNK - 1, (HCHUNK - 1) % 2)
    plsc.subcore_barrier()
    pltpu.sync_copy(accum.at[pl.ds(row0, ROWS_PER_SUB)],
                    S.at[c, pl.ds(row0, ROWS_PER_SUB)])

    pltpu.sync_copy(flag, flag_s)
    flag_v = flag_s[...]

    @pl.when(flag_v[0] > 0)
    def _deg_phase():
        zero_all()
        pltpu.sync_copy(cones, rows[0])
        pltpu.sync_copy(dstE.at[sid, pl.ds(c * DCHUNK, DCHUNK)], dst_v)
        plsc.subcore_barrier()

        def ones_pair(g, carry):
            for b in range(2):
                j = 2 * g + b
                pltpu.async_copy(rows[0], accum.at[dst_v.at[j]], ssem[b],
                                 add=True)
            for b in range(2):
                pltpu.make_async_copy(rows[0], accum.at[dst_v.at[2 * g + b]],
                                      ssem[b]).wait()
            return carry

        lax.fori_loop(0, DCHUNK // 2, ones_pair, 0)
        plsc.subcore_barrier()
        pltpu.sync_copy(accum.at[pl.ds(row0, ROWS_PER_SUB)],
                        degP.at[c, pl.ds(row0, ROWS_PER_SUB)])


_SEG_SUM = None


def _make_seg_sum():
    global _SEG_SUM
    if _SEG_SUM is not None:
        return _SEG_SUM
    mesh = plsc.VectorSubcoreMesh(core_axis_name="c", subcore_axis_name="s")
    _SEG_SUM = pl.kernel(
        _seg_sum_body,
        out_type=[jax.ShapeDtypeStruct((NCORE, NPAD, DH), jnp.float32),
                  jax.ShapeDtypeStruct((NCORE, NPAD, DH), jnp.float32)],
        mesh=mesh,
        scratch_types=[
            pltpu.VMEM((HCHUNK, CH), jnp.int32),
            pltpu.VMEM((HCHUNK, CH), jnp.int32),
            pltpu.VMEM((CH, DH), jnp.float32),
            pltpu.VMEM((CH, DH), jnp.float32),
            pltpu.SemaphoreType.DMA,
            pltpu.SemaphoreType.DMA,
            pltpu.SemaphoreType.DMA,
            pltpu.SemaphoreType.DMA,
            pltpu.VMEM((16,), jnp.int32),
            pltpu.VMEM_SHARED((NPAD, DH), jnp.float32),
        ])
    return _SEG_SUM


_TC_R = 2048


def _layer1_body(s_ref, deg_ref, x_ref, wl_ref, wr_ref, b_ref, o_ref):
    deg = deg_ref[0, :, 0:1] + deg_ref[1, :, 0:1]
    inv = 1.0 / jnp.maximum(deg, 1.0)
    mean = jnp.concatenate([s_ref[0], s_ref[1]], axis=1) * inv
    h = jnp.dot(mean, wl_ref[...], preferred_element_type=jnp.float32)
    h = h + jnp.dot(x_ref[...], wr_ref[...], preferred_element_type=jnp.float32)
    h = jnp.maximum(h + b_ref[...], 0.0)
    o_ref[0] = h[:, :DH]
    o_ref[1] = h[:, DH:]


def _layer2_body(s_ref, deg_ref, h_ref, wl_ref, wr_ref, b_ref,
                 wo_ref, bo_ref, o_ref):
    deg = deg_ref[0, :, 0:1] + deg_ref[1, :, 0:1]
    inv = 1.0 / jnp.maximum(deg, 1.0)
    mean = jnp.concatenate([s_ref[0], s_ref[1]], axis=1) * inv
    h1 = jnp.concatenate([h_ref[0], h_ref[1]], axis=1)
    h2 = jnp.dot(mean, wl_ref[...], preferred_element_type=jnp.float32)
    h2 = h2 + jnp.dot(h1, wr_ref[...], preferred_element_type=jnp.float32)
    h2 = h2 + b_ref[...]
    o_ref[...] = (jnp.dot(h2, wo_ref[...], preferred_element_type=jnp.float32)
                  + bo_ref[...])


def _tc_layer1(S1, degP, x_pad, W1l, W1r, b1):
    grid = NPAD // _TC_R
    return pl.pallas_call(
        _layer1_body,
        grid=(grid,),
        in_specs=[
            pl.BlockSpec((NCORE, _TC_R, DH), lambda i: (0, i, 0)),
            pl.BlockSpec((NCORE, _TC_R, DH), lambda i: (0, i, 0)),
            pl.BlockSpec((_TC_R, D), lambda i: (i, 0)),
            pl.BlockSpec((D, D), lambda i: (0, 0)),
            pl.BlockSpec((D, D), lambda i: (0, 0)),
            pl.BlockSpec((1, D), lambda i: (0, 0)),
        ],
        out_specs=pl.BlockSpec((NCORE, _TC_R, DH), lambda i: (0, i, 0)),
        out_shape=jax.ShapeDtypeStruct((NCORE, NPAD, DH), jnp.float32),
    )(S1, degP, x_pad, W1l, W1r, b1)


def _tc_layer2(S2, degP, h1s, W2l, W2r, b2, Wout, bout):
    grid = NPAD // _TC_R
    return pl.pallas_call(
        _layer2_body,
        grid=(grid,),
        in_specs=[
            pl.BlockSpec((NCORE, _TC_R, DH), lambda i: (0, i, 0)),
            pl.BlockSpec((NCORE, _TC_R, DH), lambda i: (0, i, 0)),
            pl.BlockSpec((NCORE, _TC_R, DH), lambda i: (0, i, 0)),
            pl.BlockSpec((D, D), lambda i: (0, 0)),
            pl.BlockSpec((D, D), lambda i: (0, 0)),
            pl.BlockSpec((1, D), lambda i: (0, 0)),
            pl.BlockSpec((D, 128), lambda i: (0, 0)),
            pl.BlockSpec((1, 128), lambda i: (0, 0)),
        ],
        out_specs=pl.BlockSpec((_TC_R, 128), lambda i: (i, 0)),
        out_shape=jax.ShapeDtypeStruct((NPAD, 128), jnp.float32),
    )(S2, degP, h1s, W2l, W2r, b2, Wout, bout)


def kernel(x, edge_index, W1l, W1r, b1, W2l, W2r, b2, Wout, bout):
    src = edge_index[0]
    dst = edge_index[1]

    x_pad = jnp.pad(x, ((0, NPAD - N), (0, 0)))
    xs = jnp.stack([x_pad[:, :DH], x_pad[:, DH:]])

    pad_e = NSUB * EPS - NE
    srcp = jnp.pad(src, (0, pad_e))
    dstp = jnp.pad(dst, (0, pad_e), constant_values=N)
    srcE = srcp.reshape(NSUB, NCHUNK, CH)
    dstE = dstp.reshape(NSUB, NCHUNK, CH)

    z128 = jnp.zeros((ZR, DH), jnp.float32)
    cones = jnp.ones((CH, DH), jnp.float32)

    one_f = jnp.ones((16,), jnp.int32)
    zero_f = jnp.zeros((16,), jnp.int32)
    S1, degP = _make_seg_sum()(xs, srcE, dstE, z128, cones, one_f)
    h1s = _tc_layer1(S1, degP, x_pad, W1l, W1r, b1.reshape(1, D))

    S2, _ = _make_seg_sum()(h1s, srcE, dstE, z128, cones, zero_f)
    out = _tc_layer2(S2, degP, h1s, W2l, W2r, b2.reshape(1, D),
                     Wout, bout.reshape(1, 128))
    return out[:N]
